# baseline (speedup 1.0000x reference)
import functools

import jax
import jax.numpy as jnp
from jax.experimental import pallas as pl
from jax.experimental.pallas import tpu as pltpu


def _fused_body(nch, chunk, mchunk, s_per_b, x_ref, key_ref, w1_ref, w2_ref,
                out_ref, idx_ref, rsim_ref, xc_ref, acc_ref, major_ref):
    i = pl.program_id(0)
    nch_per_b = s_per_b // chunk
    eps = 1e-12

    @pl.when(i == 0)
    def _init():
        acc_ref[...] = jnp.zeros_like(acc_ref)

    @pl.when(i < nch)
    def _accumulate():
        x = x_ref[...]
        xc_ref[pl.ds(i * chunk, chunk), :] = x.astype(xc_ref.dtype)
        b = i // nch_per_b
        acc_ref[pl.ds(b, 1), :] += jnp.sum(x, axis=0, keepdims=True)

    @pl.when(i == nch)
    def _route():
        bsz = idx_ref.shape[0]
        na = key_ref.shape[0]
        xm = acc_ref[0:bsz, :] * (1.0 / s_per_b)
        ak = key_ref[...]
        akn = ak * jax.lax.rsqrt(jnp.maximum(jnp.sum(ak * ak, axis=1, keepdims=True), eps))
        xn = xm * jax.lax.rsqrt(jnp.maximum(jnp.sum(xm * xm, axis=1, keepdims=True), eps))
        sim = jnp.dot(xn, akn.T, preferred_element_type=jnp.float32,
                      precision=jax.lax.Precision.HIGHEST)
        col = jax.lax.broadcasted_iota(jnp.int32, (bsz, na), 1)
        rowmax = jnp.max(sim, axis=1, keepdims=True)
        idx = jnp.min(jnp.where(sim == rowmax, col, na), axis=1, keepdims=True)
        counts = jnp.sum((idx == col).astype(jnp.int32), axis=0, keepdims=True)
        cmax = jnp.max(counts)
        major = jnp.min(jnp.where(counts == cmax, col[:1, :], na))
        idx_ref[...] = jnp.full_like(idx_ref, major)
        rsim_ref[...] = jnp.full_like(
            rsim_ref,
            jnp.sum(jnp.where(col[:1, :] == major, jnp.sum(sim, axis=0, keepdims=True), 0.0)) / bsz,
        )
        major_ref[0] = major

    @pl.when(i > nch)
    def _mlp():
        j = i - nch - 1
        mj = major_ref[0]
        w1 = w1_ref[mj]
        w2 = w2_ref[mj]
        sub = mchunk // 4
        for t in range(4):
            xb = xc_ref[pl.ds(j * mchunk + t * sub, sub), :]
            h = jnp.maximum(jnp.dot(xb, w1, preferred_element_type=jnp.float32), 0.0)
            a = jnp.maximum(
                jnp.dot(h.astype(jnp.bfloat16), w2, preferred_element_type=jnp.float32), 0.0)
            out_ref[pl.ds(t * sub, sub), :] = xb.astype(jnp.float32) + a


def kernel(x_embed, adapter_key, W1, W2, interpret=False):
    B, S, C = x_embed.shape
    NA = adapter_key.shape[0]
    H = W1.shape[2]
    rows = B * S
    chunk = 512
    mchunk = 512
    nch = rows // chunk
    nmch = rows // mchunk

    out, idx_full, rsim = pl.pallas_call(
        functools.partial(_fused_body, nch, chunk, mchunk, S),
        grid=(nch + 1 + nmch,),
        in_specs=[
            pl.BlockSpec((chunk, C), lambda i: (jnp.minimum(i, nch - 1), 0)),
            pl.BlockSpec((NA, C), lambda i: (0, 0)),
            pl.BlockSpec((NA, C, H), lambda i: (0, 0, 0)),
            pl.BlockSpec((NA, H, C), lambda i: (0, 0, 0)),
        ],
        out_specs=[
            pl.BlockSpec((mchunk, C), lambda i: (jnp.maximum(i - nch - 1, 0), 0)),
            pl.BlockSpec((B, 1), lambda i: (0, 0)),
            pl.BlockSpec((1, 1), lambda i: (0, 0)),
        ],
        out_shape=[
            jax.ShapeDtypeStruct((rows, C), jnp.float32),
            jax.ShapeDtypeStruct((B, 1), jnp.int32),
            jax.ShapeDtypeStruct((1, 1), jnp.float32),
        ],
        scratch_shapes=[
            pltpu.VMEM((rows, C), jnp.bfloat16),
            pltpu.VMEM((8, C), jnp.float32),
            pltpu.SMEM((1,), jnp.int32),
        ],
        interpret=interpret,
    )(x_embed.reshape(rows, C), adapter_key,
      W1.astype(jnp.bfloat16), W2.astype(jnp.bfloat16))
    return idx_full, rsim.reshape(()), out.reshape(B, S, C)


# speedup vs baseline: 1.0972x; 1.0972x over previous
import functools

import jax
import jax.numpy as jnp
from jax.experimental import pallas as pl
from jax.experimental.pallas import tpu as pltpu


def _fused_body(nch, chunk, mchunk, s_per_b, x_ref, key_ref, w1_ref, w2_ref,
                out_ref, idx_ref, rsim_ref, xc_ref, acc_ref, major_ref):
    i = pl.program_id(0)
    nch_per_b = s_per_b // chunk
    eps = 1e-12

    @pl.when(i == 0)
    def _init():
        acc_ref[...] = jnp.zeros_like(acc_ref)

    @pl.when(i < nch)
    def _accumulate():
        x = x_ref[...]
        xc_ref[pl.ds(i * chunk, chunk), :] = x.astype(xc_ref.dtype)
        b = i // nch_per_b
        acc_ref[pl.ds(b, 1), :] += jnp.sum(x, axis=0, keepdims=True)

    @pl.when(i == nch)
    def _route():
        bsz = idx_ref.shape[0]
        na = key_ref.shape[0]
        xm = acc_ref[0:bsz, :] * (1.0 / s_per_b)
        ak = key_ref[...]
        akn = ak * jax.lax.rsqrt(jnp.maximum(jnp.sum(ak * ak, axis=1, keepdims=True), eps))
        xn = xm * jax.lax.rsqrt(jnp.maximum(jnp.sum(xm * xm, axis=1, keepdims=True), eps))
        sim = jnp.dot(xn, akn.T, preferred_element_type=jnp.float32,
                      precision=jax.lax.Precision.HIGHEST)
        col = jax.lax.broadcasted_iota(jnp.int32, (bsz, na), 1)
        rowmax = jnp.max(sim, axis=1, keepdims=True)
        idx = jnp.min(jnp.where(sim == rowmax, col, na), axis=1, keepdims=True)
        counts = jnp.sum((idx == col).astype(jnp.int32), axis=0, keepdims=True)
        cmax = jnp.max(counts)
        major = jnp.min(jnp.where(counts == cmax, col[:1, :], na))
        idx_ref[...] = jnp.full_like(idx_ref, major)
        rsim_ref[...] = jnp.full_like(
            rsim_ref,
            jnp.sum(jnp.where(col[:1, :] == major, jnp.sum(sim, axis=0, keepdims=True), 0.0)) / bsz,
        )
        major_ref[0] = major

    @pl.when(i > nch)
    def _mlp():
        j = i - nch - 1
        mj = major_ref[0]
        w1 = w1_ref[mj]
        w2 = w2_ref[mj]
        sub = mchunk // 2
        for t in range(2):
            xb = xc_ref[pl.ds(j * mchunk + t * sub, sub), :]
            h = jnp.maximum(jnp.dot(xb, w1, preferred_element_type=jnp.float32), 0.0)
            ap = jnp.dot(h.astype(jnp.bfloat16), w2, preferred_element_type=jnp.float32)
            xf = xb.astype(jnp.float32)
            out_ref[pl.ds(t * sub, sub), :] = jnp.maximum(xf + ap, xf)


def kernel(x_embed, adapter_key, W1, W2, interpret=False):
    B, S, C = x_embed.shape
    NA = adapter_key.shape[0]
    H = W1.shape[2]
    rows = B * S
    chunk = 512
    mchunk = 512
    nch = rows // chunk
    nmch = rows // mchunk

    out, idx_full, rsim = pl.pallas_call(
        functools.partial(_fused_body, nch, chunk, mchunk, S),
        grid=(nch + 1 + nmch,),
        in_specs=[
            pl.BlockSpec((chunk, C), lambda i: (jnp.minimum(i, nch - 1), 0)),
            pl.BlockSpec((NA, C), lambda i: (0, 0)),
            pl.BlockSpec((NA, C, H), lambda i: (0, 0, 0)),
            pl.BlockSpec((NA, H, C), lambda i: (0, 0, 0)),
        ],
        out_specs=[
            pl.BlockSpec((mchunk, C), lambda i: (jnp.maximum(i - nch - 1, 0), 0)),
            pl.BlockSpec((B, 1), lambda i: (0, 0)),
            pl.BlockSpec((1, 1), lambda i: (0, 0)),
        ],
        out_shape=[
            jax.ShapeDtypeStruct((rows, C), jnp.float32),
            jax.ShapeDtypeStruct((B, 1), jnp.int32),
            jax.ShapeDtypeStruct((1, 1), jnp.float32),
        ],
        scratch_shapes=[
            pltpu.VMEM((rows, C), jnp.bfloat16),
            pltpu.VMEM((8, C), jnp.float32),
            pltpu.SMEM((1,), jnp.int32),
        ],
        interpret=interpret,
    )(x_embed.reshape(rows, C), adapter_key,
      W1.astype(jnp.bfloat16), W2.astype(jnp.bfloat16))
    return idx_full, rsim.reshape(()), out.reshape(B, S, C)


# speedup vs baseline: 1.3511x; 1.2314x over previous
import functools

import jax
import jax.numpy as jnp
from jax.experimental import pallas as pl
from jax.experimental.pallas import tpu as pltpu


def _fused_body(nblk, blk, s_per_b, x_ref, key_ref, w1_ref, w2_ref,
                out_ref, idx_ref, rsim_ref, acc_ref, major_ref):
    i = pl.program_id(0)
    blk_per_b = s_per_b // blk
    eps = 1e-12

    @pl.when(i == 0)
    def _init():
        acc_ref[...] = jnp.zeros_like(acc_ref)

    @pl.when(i < nblk)
    def _accumulate():
        b = i // blk_per_b
        acc_ref[pl.ds(b, 1), :] += jnp.sum(x_ref[...], axis=0, keepdims=True)

    @pl.when(i == nblk)
    def _route():
        bsz = idx_ref.shape[0]
        na = key_ref.shape[0]
        xm = acc_ref[0:bsz, :] * (1.0 / s_per_b)
        ak = key_ref[...]
        akn = ak * jax.lax.rsqrt(jnp.maximum(jnp.sum(ak * ak, axis=1, keepdims=True), eps))
        xn = xm * jax.lax.rsqrt(jnp.maximum(jnp.sum(xm * xm, axis=1, keepdims=True), eps))
        sim = jnp.dot(xn, akn.T, preferred_element_type=jnp.float32,
                      precision=jax.lax.Precision.HIGHEST)
        col = jax.lax.broadcasted_iota(jnp.int32, (bsz, na), 1)
        rowmax = jnp.max(sim, axis=1, keepdims=True)
        idx = jnp.min(jnp.where(sim == rowmax, col, na), axis=1, keepdims=True)
        counts = jnp.sum((idx == col).astype(jnp.int32), axis=0, keepdims=True)
        cmax = jnp.max(counts)
        major = jnp.min(jnp.where(counts == cmax, col[:1, :], na))
        idx_ref[...] = jnp.full_like(idx_ref, major)
        rsim_ref[...] = jnp.full_like(
            rsim_ref,
            jnp.sum(jnp.where(col[:1, :] == major, jnp.sum(sim, axis=0, keepdims=True), 0.0)) / bsz,
        )
        major_ref[0] = major

    @pl.when(i > nblk)
    def _mlp():
        mj = major_ref[0]
        w1 = w1_ref[mj]
        w2 = w2_ref[mj]
        sub = 512
        for t in range(blk // sub):
            xs = x_ref[t * sub:(t + 1) * sub, :]
            h = jnp.maximum(jnp.dot(xs, w1, preferred_element_type=jnp.float32), 0.0)
            ap = jnp.dot(h, w2, preferred_element_type=jnp.float32)
            out_ref[t * sub:(t + 1) * sub, :] = jnp.maximum(xs + ap, xs)


def kernel(x_embed, adapter_key, W1, W2, interpret=False):
    B, S, C = x_embed.shape
    NA = adapter_key.shape[0]
    H = W1.shape[2]
    rows = B * S
    blk = 4096
    nblk = rows // blk

    out, idx_full, rsim = pl.pallas_call(
        functools.partial(_fused_body, nblk, blk, S),
        grid=(2 * nblk + 1,),
        in_specs=[
            pl.BlockSpec((blk, C), lambda i: (
                jnp.where(i <= nblk, jnp.minimum(i, nblk - 1), i - nblk - 1), 0)),
            pl.BlockSpec((NA, C), lambda i: (0, 0)),
            pl.BlockSpec((NA, C, H), lambda i: (0, 0, 0)),
            pl.BlockSpec((NA, H, C), lambda i: (0, 0, 0)),
        ],
        out_specs=[
            pl.BlockSpec((blk, C), lambda i: (jnp.maximum(i - nblk - 1, 0), 0)),
            pl.BlockSpec((B, 1), lambda i: (0, 0)),
            pl.BlockSpec((1, 1), lambda i: (0, 0)),
        ],
        out_shape=[
            jax.ShapeDtypeStruct((rows, C), jnp.float32),
            jax.ShapeDtypeStruct((B, 1), jnp.int32),
            jax.ShapeDtypeStruct((1, 1), jnp.float32),
        ],
        scratch_shapes=[
            pltpu.VMEM((8, C), jnp.float32),
            pltpu.SMEM((1,), jnp.int32),
        ],
        interpret=interpret,
    )(x_embed.reshape(rows, C), adapter_key, W1, W2)
    return idx_full, rsim.reshape(()), out.reshape(B, S, C)
